# baseline (speedup 1.0000x reference)
import functools

import jax
import jax.numpy as jnp
from jax import lax
from jax.experimental import pallas as pl
from jax.experimental.pallas import tpu as pltpu
from jax.experimental.pallas import tpu_sc as plsc

F = 128
EB = 112
NT = 16
NC = 2

_SC_MESH = dict(core_axis_name="c", subcore_axis_name="s",
                num_cores=NC, num_subcores=NT)


def _make_deg(npad, nb):
    rpt = npad // NT

    @functools.partial(
        pl.kernel,
        out_type=jax.ShapeDtypeStruct((npad, F), jnp.float32),
        mesh=plsc.VectorSubcoreMesh(**_SC_MESH),
        scratch_types=[
            pltpu.VMEM((nb, EB), jnp.int32),
            pltpu.VMEM((EB, F), jnp.float32),
            pltpu.VMEM_SHARED((npad, F), jnp.float32),
            pltpu.SemaphoreType.DMA,
        ],
    )
    def deg(dstb, onesz, deg_out, dstv, onesbuf, acc_sh, dsem):
        c = lax.axis_index("c")
        s = lax.axis_index("s")
        rows = pl.ds(s * rpt, rpt)
        pltpu.sync_copy(dstb.at[s], dstv)
        pltpu.sync_copy(onesz.at[pl.ds(0, EB)], onesbuf)
        for r in range(rpt // EB):
            pltpu.sync_copy(onesz.at[pl.ds(EB, EB)],
                            acc_sh.at[pl.ds(s * rpt + r * EB, EB)])
        plsc.subcore_barrier()
        descs = [pltpu.async_copy(onesbuf, acc_sh.at[dstv.at[b]], dsem,
                                  add=True) for b in range(nb)]
        for d in descs:
            d.wait()
        plsc.subcore_barrier()

        @pl.when(c == 0)
        def _():
            pltpu.sync_copy(acc_sh.at[rows], deg_out.at[rows])

    return deg


def _make_agg(npad, nb, cpc, with_u):
    ctot = NC * cpc + (2 if with_u else 0)
    rpt = npad // NT
    nbh = (nb + 1) // 2

    @functools.partial(
        pl.kernel,
        out_type=jax.ShapeDtypeStruct((ctot * npad, F), jnp.float32),
        mesh=plsc.VectorSubcoreMesh(**_SC_MESH),
        scratch_types=[
            pltpu.VMEM((6, EB), jnp.int32),
            pltpu.VMEM((6, EB), jnp.int32),
            pltpu.VMEM((3, EB, F), jnp.float32),
            pltpu.VMEM_SHARED((npad, F), jnp.float32),
        ] + [pltpu.SemaphoreType.DMA] * 10,
    )
    def agg(htflat, gidx, dstb, *rest):
        if with_u:
            srcb, ugidx = rest[:2]
            rest = rest[2:]
        (outflat, gidxr, didxr, gbuf, acc_sh,
         si0, si1, si2, si3, sg0, sg1, sg2, ss0, ss1, ss2) = rest
        semi = [si0, si1, si2, si3]
        semg = [sg0, sg1, sg2]
        sems = [ss0, ss1, ss2]
        c = lax.axis_index("c")
        s = lax.axis_index("s")
        rows = pl.ds(s * rpt, rpt)

        def edge_pipeline(gidx_at, sidx_at, nblk):
            descs_i = [None] * 6
            descs_g = [None] * 3
            descs_s = [None] * 3

            def issue_idx(b):
                k = b % 6
                descs_i[k] = (
                    pltpu.async_copy(gidx_at(b), gidxr.at[k], semi[k % 4]),
                    pltpu.async_copy(sidx_at(b), didxr.at[k], semi[k % 4]))

            def issue_gather(b):
                k3, k6 = b % 3, b % 6
                descs_i[k6][0].wait()
                descs_i[k6][1].wait()
                descs_g[k3] = pltpu.async_copy(
                    htflat.at[gidxr.at[k6]], gbuf.at[k3], semg[k3])

            for b in range(min(2, nblk)):
                issue_idx(b)
            if nblk > 0:
                issue_gather(0)
            for b in range(nblk):
                k3 = b % 3
                if b + 1 < nblk:
                    if b + 1 >= 3:
                        descs_s[(b + 1) % 3].wait()
                    issue_gather(b + 1)
                descs_g[k3].wait()
                descs_s[k3] = pltpu.async_copy(
                    gbuf.at[k3], acc_sh.at[didxr.at[b % 6]], sems[k3],
                    add=True)
                if b + 2 < nblk:
                    issue_idx(b + 2)
            for b in range(max(0, nblk - 3), nblk):
                descs_s[b % 3].wait()

        for ci in range(cpc):
            chunk = c * cpc + ci
            tbase = chunk * npad
            pltpu.sync_copy(htflat.at[pl.ds(tbase + s * rpt, rpt)],
                            acc_sh.at[rows])
            plsc.subcore_barrier()
            goff = (chunk * NT + s) * nb
            edge_pipeline(
                lambda b, go=goff: gidx.at[pl.ds((go + b) * EB, EB)],
                lambda b: dstb.at[pl.ds((s * nb + b) * EB, EB)], nb)
            plsc.subcore_barrier()
            pltpu.sync_copy(acc_sh.at[rows],
                            outflat.at[pl.ds(tbase + s * rpt, rpt)])
            plsc.subcore_barrier()
        if with_u:
            ubase = (NC * cpc + c) * npad
            pltpu.sync_copy(htflat.at[pl.ds(ubase + s * rpt, rpt)],
                            acc_sh.at[rows])
            plsc.subcore_barrier()
            ub0 = s * nb + c * nbh
            edge_pipeline(
                lambda b: ugidx.at[pl.ds((ub0 + b) * EB, EB)],
                lambda b: srcb.at[pl.ds((ub0 + b) * EB, EB)], nbh - 1)
            bb = c * nbh + nbh - 1

            @pl.when(bb < nb)
            def _():
                pltpu.sync_copy(ugidx.at[pl.ds((s * nb + bb) * EB, EB)],
                                gidxr.at[0])
                pltpu.sync_copy(srcb.at[pl.ds((s * nb + bb) * EB, EB)],
                                didxr.at[0])
                pltpu.async_copy(htflat.at[gidxr.at[0]], gbuf.at[0],
                                 semg[0]).wait()
                pltpu.sync_copy(gbuf.at[0], acc_sh.at[didxr.at[0]],
                                add=True)

            plsc.subcore_barrier()
            pltpu.sync_copy(acc_sh.at[rows],
                            outflat.at[pl.ds(ubase + s * rpt, rpt)])

    return agg


def _scale_call(xp, cnt_col, npad, cin):
    rb = 512
    grid = (npad // rb, cin + 2)

    def body(x_ref, g_ref, o_ref, d_ref):
        j = pl.program_id(1)
        dis = lax.rsqrt(1.0 + g_ref[...])

        @pl.when(j < cin)
        def _():
            o_ref[0] = x_ref[...] * dis

        @pl.when(j == cin)
        def _():
            lane = lax.broadcasted_iota(jnp.int32, (rb, F), 1)
            o_ref[0] = jnp.where(lane == 0, dis, 0.0)

        @pl.when(j == cin + 1)
        def _():
            o_ref[0] = jnp.zeros((rb, F), jnp.float32)

        @pl.when(j == 0)
        def _():
            d_ref[...] = dis

    return pl.pallas_call(
        body,
        grid=grid,
        in_specs=[
            pl.BlockSpec((rb, F), lambda i, j: (i, jnp.minimum(j, cin - 1))),
            pl.BlockSpec((rb, 1), lambda i, j: (i, 0)),
        ],
        out_specs=[
            pl.BlockSpec((1, rb, F), lambda i, j: (j, i, 0)),
            pl.BlockSpec((rb, 1), lambda i, j: (i, 0)),
        ],
        out_shape=[jax.ShapeDtypeStruct((cin + 2, npad, F), jnp.float32),
                   jax.ShapeDtypeStruct((npad, 1), jnp.float32)],
    )(xp, cnt_col)


def _layer_call(s3d, dis_col, W, b_row, npad):
    cin = s3d.shape[0]
    cout = W.shape[1] // F
    rb = 512
    grid = (npad // rb, cout)

    def body(s_ref, d_ref, w_ref, b_ref, o_ref):
        dis = d_ref[...]
        acc = jnp.zeros((rb, F), jnp.float32)
        for k in range(cin):
            acc = acc + jnp.dot(
                (dis * s_ref[k]).astype(jnp.bfloat16),
                w_ref[pl.ds(k * F, F), :].astype(jnp.bfloat16),
                preferred_element_type=jnp.float32)
        h = jnp.maximum(acc + b_ref[...], 0.0)
        o_ref[0] = h * dis

    return pl.pallas_call(
        body,
        grid=grid,
        in_specs=[
            pl.BlockSpec((cin, rb, F), lambda i, j: (0, i, 0)),
            pl.BlockSpec((rb, 1), lambda i, j: (i, 0)),
            pl.BlockSpec((W.shape[0], F), lambda i, j: (0, j)),
            pl.BlockSpec((1, F), lambda i, j: (0, j)),
        ],
        out_specs=pl.BlockSpec((1, rb, F), lambda i, j: (j, i, 0)),
        out_shape=jax.ShapeDtypeStruct((cout, npad, F), jnp.float32),
    )(s3d, dis_col, W, b_row)


def _final_call(s3d, dis_col, su_a, su_b, W2, b2_row, W3, b3_row, npad,
                nreal):
    cin = s3d.shape[0]
    hid = W2.shape[1]
    ncls = W3.shape[1]
    rb = 512
    grid = (npad // rb,)

    def body(s_ref, d_ref, ua_ref, ub_ref, w2_ref, b2_ref, w3_ref, b3_ref,
             o_ref, vacc):
        i = pl.program_id(0)
        dis = d_ref[...]
        acc = jnp.zeros((rb, hid), jnp.float32)
        for k in range(cin):
            acc = acc + jnp.dot(
                (dis * s_ref[k]).astype(jnp.bfloat16),
                w2_ref[pl.ds(k * F, F), :].astype(jnp.bfloat16),
                preferred_element_type=jnp.float32)
        h3 = jnp.maximum(acc + b2_ref[...], 0.0)
        g = i * rb + lax.broadcasted_iota(jnp.int32, (rb, 1), 0)
        wv = jnp.where(g < nreal, dis * (ua_ref[...] + ub_ref[...]), 0.0)
        contrib = lax.dot_general(wv, h3, (((0,), (0,)), ((), ())),
                                  preferred_element_type=jnp.float32)

        @pl.when(i == 0)
        def _():
            vacc[...] = contrib

        @pl.when(i > 0)
        def _():
            vacc[...] = vacc[...] + contrib

        @pl.when(i == pl.num_programs(0) - 1)
        def _():
            o_ref[...] = (jnp.dot(vacc[...], w3_ref[...],
                                  preferred_element_type=jnp.float32)
                          * (1.0 / nreal) + b3_ref[...])

    return pl.pallas_call(
        body,
        grid=grid,
        in_specs=[
            pl.BlockSpec((cin, rb, F), lambda i: (0, i, 0)),
            pl.BlockSpec((rb, 1), lambda i: (i, 0)),
            pl.BlockSpec((rb, 1), lambda i: (i, 0)),
            pl.BlockSpec((rb, 1), lambda i: (i, 0)),
            pl.BlockSpec((W2.shape[0], hid), lambda i: (0, 0)),
            pl.BlockSpec((1, hid), lambda i: (0, 0)),
            pl.BlockSpec((hid, ncls), lambda i: (0, 0)),
            pl.BlockSpec((1, ncls), lambda i: (0, 0)),
        ],
        out_specs=pl.BlockSpec((1, ncls), lambda i: (0, 0)),
        out_shape=jax.ShapeDtypeStruct((1, ncls), jnp.float32),
        scratch_shapes=[pltpu.VMEM((1, hid), jnp.float32)],
    )(s3d, dis_col, su_a, su_b, W2, b2_row, W3, b3_row)


def kernel(X, edge_list, W0, b0, W1, b1, W2, b2, W3, b3):
    n, din = X.shape
    hid = W1.shape[0]
    npad = 512 * ((n + 511) // 512)
    e = edge_list.shape[1]
    unit = NT * EB
    epad = unit * ((e + unit - 1) // unit)
    nb = epad // unit

    src = edge_list[0].astype(jnp.int32)
    dst = edge_list[1].astype(jnp.int32)
    padn = epad - e
    if padn:
        pad_idx = n + (jnp.arange(padn, dtype=jnp.int32) % (npad - n))
        src = jnp.concatenate([src, pad_idx])
        dst = jnp.concatenate([dst, pad_idx])
    srcb = src.reshape(NT, nb, EB)
    dstb = dst.reshape(NT, nb, EB)
    onesz = jnp.concatenate([jnp.ones((EB, F), jnp.float32),
                             jnp.zeros((EB, F), jnp.float32)])

    sdeg = _make_deg(npad, nb)(dstb, onesz)

    xp = jnp.pad(X, ((0, npad - n), (0, 0)))
    cin0 = din // F
    ch = hid // F
    gidx4 = (srcb[None] +
             (jnp.arange(ch, dtype=jnp.int32) * npad)[:, None, None, None])
    gidx2 = gidx4[:cin0]
    ugidx = dstb + cin0 * npad

    ht0ext, dis_col = _scale_call(xp, sdeg[:, 0:1], npad, cin0)
    s0ext = _make_agg(npad, nb, cin0 // NC, True)(
        ht0ext.reshape(-1, F), gidx2.reshape(-1), dstb.reshape(-1),
        srcb.reshape(-1), ugidx.reshape(-1))
    ht1 = _layer_call(s0ext[:cin0 * npad].reshape(cin0, npad, F), dis_col,
                      W0, b0.reshape(1, -1), npad)
    agg4 = _make_agg(npad, nb, ch // NC, False)
    dstf = dstb.reshape(-1)
    gidx4f = gidx4.reshape(-1)
    s1 = agg4(ht1.reshape(-1, F), gidx4f, dstf)
    ht2 = _layer_call(s1.reshape(ch, npad, F), dis_col, W1,
                      b1.reshape(1, -1), npad)
    s2 = agg4(ht2.reshape(-1, F), gidx4f, dstf)
    su_a = s0ext[cin0 * npad:(cin0 + 1) * npad, 0:1]
    su_b = s0ext[(cin0 + 1) * npad:, 0:1]
    return _final_call(s2.reshape(ch, npad, F), dis_col, su_a, su_b,
                       W2, b2.reshape(1, -1), W3, b3.reshape(1, -1), npad, n)


# speedup vs baseline: 1.0510x; 1.0510x over previous
import functools

import jax
import jax.numpy as jnp
from jax import lax
from jax.experimental import pallas as pl
from jax.experimental.pallas import tpu as pltpu
from jax.experimental.pallas import tpu_sc as plsc

F = 128
EB = 128
NT = 16
NC = 2

_SC_MESH = dict(core_axis_name="c", subcore_axis_name="s",
                num_cores=NC, num_subcores=NT)


def _make_deg(npad, nb):
    rpt = npad // NT

    @functools.partial(
        pl.kernel,
        out_type=jax.ShapeDtypeStruct((npad, F), jnp.float32),
        mesh=plsc.VectorSubcoreMesh(**_SC_MESH),
        scratch_types=[
            pltpu.VMEM((nb, EB), jnp.int32),
            pltpu.VMEM((EB, F), jnp.float32),
            pltpu.VMEM_SHARED((npad, F), jnp.float32),
            pltpu.SemaphoreType.DMA,
        ],
    )
    def deg(dstb, onesz, deg_out, dstv, onesbuf, acc_sh, dsem):
        c = lax.axis_index("c")
        s = lax.axis_index("s")
        rows = pl.ds(s * rpt, rpt)
        pltpu.sync_copy(dstb.at[s], dstv)
        pltpu.sync_copy(onesz.at[pl.ds(0, EB)], onesbuf)
        for r in range(rpt // EB):
            pltpu.sync_copy(onesz.at[pl.ds(EB, EB)],
                            acc_sh.at[pl.ds(s * rpt + r * EB, EB)])
        plsc.subcore_barrier()
        descs = [pltpu.async_copy(onesbuf, acc_sh.at[dstv.at[b]], dsem,
                                  add=True) for b in range(nb)]
        for d in descs:
            d.wait()
        plsc.subcore_barrier()

        @pl.when(c == 0)
        def _():
            pltpu.sync_copy(acc_sh.at[rows], deg_out.at[rows])

    return deg


def _make_agg(npad, nb, cpc, with_u):
    ctot = NC * cpc + (2 if with_u else 0)
    rpt = npad // NT
    nbh = (nb + 1) // 2

    @functools.partial(
        pl.kernel,
        out_type=jax.ShapeDtypeStruct((ctot * npad, F), jnp.float32),
        mesh=plsc.VectorSubcoreMesh(**_SC_MESH),
        scratch_types=[
            pltpu.VMEM((4, EB), jnp.int32),
            pltpu.VMEM((4, EB), jnp.int32),
            pltpu.VMEM((2, EB, F), jnp.float32),
            pltpu.VMEM_SHARED((npad, F), jnp.float32),
        ] + [pltpu.SemaphoreType.DMA] * 8,
    )
    def agg(htflat, gidx, dstb, *rest):
        if with_u:
            srcb, ugidx = rest[:2]
            rest = rest[2:]
        (outflat, gidxr, didxr, gbuf, acc_sh,
         si0, si1, si2, si3, sg0, sg1, ss0, ss1) = rest
        semi = [si0, si1, si2, si3]
        semg = [sg0, sg1]
        sems = [ss0, ss1]
        c = lax.axis_index("c")
        s = lax.axis_index("s")
        rows = pl.ds(s * rpt, rpt)

        def edge_pipeline(gidx_at, sidx_at, nblk):
            descs_i = [None] * 4
            descs_g = [None] * 2
            descs_s = [None] * 2

            def issue_idx(b):
                k = b % 4
                descs_i[k] = (
                    pltpu.async_copy(gidx_at(b), gidxr.at[k], semi[k]),
                    pltpu.async_copy(sidx_at(b), didxr.at[k], semi[k]))

            def issue_gather(b):
                k2, k4 = b % 2, b % 4
                descs_i[k4][0].wait()
                descs_i[k4][1].wait()
                descs_g[k2] = pltpu.async_copy(
                    htflat.at[gidxr.at[k4]], gbuf.at[k2], semg[k2])

            for b in range(min(2, nblk)):
                issue_idx(b)
            if nblk > 0:
                issue_gather(0)
            for b in range(nblk):
                k2 = b % 2
                if b + 1 < nblk:
                    if b >= 1:
                        descs_s[(b + 1) % 2].wait()
                    issue_gather(b + 1)
                descs_g[k2].wait()
                descs_s[k2] = pltpu.async_copy(
                    gbuf.at[k2], acc_sh.at[didxr.at[b % 4]], sems[k2],
                    add=True)
                if b + 2 < nblk:
                    issue_idx(b + 2)
            for b in range(max(0, nblk - 2), nblk):
                descs_s[b % 2].wait()

        for ci in range(cpc):
            chunk = c * cpc + ci
            tbase = chunk * npad
            pltpu.sync_copy(htflat.at[pl.ds(tbase + s * rpt, rpt)],
                            acc_sh.at[rows])
            plsc.subcore_barrier()
            goff = (chunk * NT + s) * nb
            edge_pipeline(
                lambda b, go=goff: gidx.at[pl.ds((go + b) * EB, EB)],
                lambda b: dstb.at[pl.ds((s * nb + b) * EB, EB)], nb)
            plsc.subcore_barrier()
            pltpu.sync_copy(acc_sh.at[rows],
                            outflat.at[pl.ds(tbase + s * rpt, rpt)])
            plsc.subcore_barrier()
        if with_u:
            ubase = (NC * cpc + c) * npad
            pltpu.sync_copy(htflat.at[pl.ds(ubase + s * rpt, rpt)],
                            acc_sh.at[rows])
            plsc.subcore_barrier()
            ub0 = s * nb + c * nbh
            edge_pipeline(
                lambda b: ugidx.at[pl.ds((ub0 + b) * EB, EB)],
                lambda b: srcb.at[pl.ds((ub0 + b) * EB, EB)], nbh - 1)
            bb = c * nbh + nbh - 1

            @pl.when(bb < nb)
            def _():
                pltpu.sync_copy(ugidx.at[pl.ds((s * nb + bb) * EB, EB)],
                                gidxr.at[0])
                pltpu.sync_copy(srcb.at[pl.ds((s * nb + bb) * EB, EB)],
                                didxr.at[0])
                pltpu.async_copy(htflat.at[gidxr.at[0]], gbuf.at[0],
                                 semg[0]).wait()
                pltpu.sync_copy(gbuf.at[0], acc_sh.at[didxr.at[0]],
                                add=True)

            plsc.subcore_barrier()
            pltpu.sync_copy(acc_sh.at[rows],
                            outflat.at[pl.ds(ubase + s * rpt, rpt)])

    return agg


def _scale_call(xp, cnt_col, npad, cin):
    rb = 512
    grid = (npad // rb, cin + 1)

    def body(x_ref, g_ref, o_ref, d_ref):
        j = pl.program_id(1)
        dis = lax.rsqrt(1.0 + g_ref[...])

        @pl.when(j < cin)
        def _():
            o_ref[0] = x_ref[...] * dis

        @pl.when(j == cin)
        def _():
            lane = lax.broadcasted_iota(jnp.int32, (rb, F), 1)
            o_ref[0] = jnp.where(lane == 0, dis, 0.0)

        @pl.when(j == 0)
        def _():
            d_ref[...] = dis

    return pl.pallas_call(
        body,
        grid=grid,
        in_specs=[
            pl.BlockSpec((rb, F), lambda i, j: (i, jnp.minimum(j, cin - 1))),
            pl.BlockSpec((rb, 1), lambda i, j: (i, 0)),
        ],
        out_specs=[
            pl.BlockSpec((1, rb, F), lambda i, j: (j, i, 0)),
            pl.BlockSpec((rb, 1), lambda i, j: (i, 0)),
        ],
        out_shape=[jax.ShapeDtypeStruct((cin + 1, npad, F), jnp.float32),
                   jax.ShapeDtypeStruct((npad, 1), jnp.float32)],
    )(xp, cnt_col)


def _layer_call(s3d, dis_col, W, b_row, npad, cin):
    cout = W.shape[1] // F
    rb = 512
    grid = (npad // rb, cout)

    def body(s_ref, d_ref, w_ref, b_ref, o_ref):
        j = pl.program_id(1)
        dis = d_ref[...]
        acc = jnp.zeros((rb, F), jnp.float32)
        for k in range(cin):
            acc = acc + jnp.dot(
                (dis * s_ref[k]).astype(jnp.bfloat16),
                w_ref[pl.ds(k * F, F), pl.ds(j * F, F)].astype(jnp.bfloat16),
                preferred_element_type=jnp.float32)
        h = jnp.maximum(acc + b_ref[...], 0.0)
        o_ref[0] = h * dis

    return pl.pallas_call(
        body,
        grid=grid,
        in_specs=[
            pl.BlockSpec((cin, rb, F), lambda i, j: (0, i, 0)),
            pl.BlockSpec((rb, 1), lambda i, j: (i, 0)),
            pl.BlockSpec((W.shape[0], W.shape[1]), lambda i, j: (0, 0)),
            pl.BlockSpec((1, F), lambda i, j: (0, j)),
        ],
        out_specs=pl.BlockSpec((1, rb, F), lambda i, j: (j, i, 0)),
        out_shape=jax.ShapeDtypeStruct((cout, npad, F), jnp.float32),
    )(s3d, dis_col, W, b_row)


def _final_call(s3d, dis_col, su_a, su_b, W2, b2_row, W3, b3_row, npad,
                nreal):
    cin = s3d.shape[0]
    hid = W2.shape[1]
    ncls = W3.shape[1]
    rb = 512
    grid = (npad // rb,)

    def body(s_ref, d_ref, ua_ref, ub_ref, w2_ref, b2_ref, w3_ref, b3_ref,
             o_ref, vacc):
        i = pl.program_id(0)
        dis = d_ref[...]
        acc = jnp.zeros((rb, hid), jnp.float32)
        for k in range(cin):
            acc = acc + jnp.dot(
                (dis * s_ref[k]).astype(jnp.bfloat16),
                w2_ref[pl.ds(k * F, F), :].astype(jnp.bfloat16),
                preferred_element_type=jnp.float32)
        h3 = jnp.maximum(acc + b2_ref[...], 0.0)
        g = i * rb + lax.broadcasted_iota(jnp.int32, (rb, 1), 0)
        wv = jnp.where(g < nreal,
                       dis * (ua_ref[...] + ub_ref[...] - dis), 0.0)
        contrib = lax.dot_general(wv, h3, (((0,), (0,)), ((), ())),
                                  preferred_element_type=jnp.float32)

        @pl.when(i == 0)
        def _():
            vacc[...] = contrib

        @pl.when(i > 0)
        def _():
            vacc[...] = vacc[...] + contrib

        @pl.when(i == pl.num_programs(0) - 1)
        def _():
            o_ref[...] = (jnp.dot(vacc[...], w3_ref[...],
                                  preferred_element_type=jnp.float32)
                          * (1.0 / nreal) + b3_ref[...])

    return pl.pallas_call(
        body,
        grid=grid,
        in_specs=[
            pl.BlockSpec((cin, rb, F), lambda i: (0, i, 0)),
            pl.BlockSpec((rb, 1), lambda i: (i, 0)),
            pl.BlockSpec((rb, 1), lambda i: (i, 0)),
            pl.BlockSpec((rb, 1), lambda i: (i, 0)),
            pl.BlockSpec((W2.shape[0], hid), lambda i: (0, 0)),
            pl.BlockSpec((1, hid), lambda i: (0, 0)),
            pl.BlockSpec((hid, ncls), lambda i: (0, 0)),
            pl.BlockSpec((1, ncls), lambda i: (0, 0)),
        ],
        out_specs=pl.BlockSpec((1, ncls), lambda i: (0, 0)),
        out_shape=jax.ShapeDtypeStruct((1, ncls), jnp.float32),
        scratch_shapes=[pltpu.VMEM((1, hid), jnp.float32)],
    )(s3d, dis_col, su_a, su_b, W2, b2_row, W3, b3_row)


def kernel(X, edge_list, W0, b0, W1, b1, W2, b2, W3, b3):
    n, din = X.shape
    hid = W1.shape[0]
    npad = 512 * ((n + 511) // 512)
    e = edge_list.shape[1]
    unit = NT * EB
    epad = unit * ((e + unit - 1) // unit)
    nb = epad // unit

    src = edge_list[0].astype(jnp.int32)
    dst = edge_list[1].astype(jnp.int32)
    padn = epad - e
    if padn:
        pad_idx = n + (jnp.arange(padn, dtype=jnp.int32) % (npad - n))
        src = jnp.concatenate([src, pad_idx])
        dst = jnp.concatenate([dst, pad_idx])
    srcb = src.reshape(NT, nb, EB)
    dstb = dst.reshape(NT, nb, EB)
    onesz = jnp.concatenate([jnp.ones((EB, F), jnp.float32),
                             jnp.zeros((EB, F), jnp.float32)])

    sdeg = _make_deg(npad, nb)(dstb, onesz)

    xp = jnp.pad(X, ((0, npad - n), (0, 0)))
    cin0 = din // F
    ch = hid // F
    gidx4 = (srcb[None] +
             (jnp.arange(ch, dtype=jnp.int32) * npad)[:, None, None, None])
    gidx2 = gidx4[:cin0]
    ugidx = dstb + cin0 * npad

    ht0ext, dis_col = _scale_call(xp, sdeg[:, 0:1], npad, cin0)
    s0ext = _make_agg(npad, nb, cin0 // NC, True)(
        ht0ext.reshape(-1, F), gidx2.reshape(-1), dstb.reshape(-1),
        srcb.reshape(-1), ugidx.reshape(-1))
    ht1 = _layer_call(s0ext.reshape(cin0 + 2, npad, F), dis_col,
                      W0, b0.reshape(1, -1), npad, cin0)
    agg4 = _make_agg(npad, nb, ch // NC, False)
    dstf = dstb.reshape(-1)
    gidx4f = gidx4.reshape(-1)
    s1 = agg4(ht1.reshape(-1, F), gidx4f, dstf)
    ht2 = _layer_call(s1.reshape(ch, npad, F), dis_col, W1,
                      b1.reshape(1, -1), npad, ch)
    s2 = agg4(ht2.reshape(-1, F), gidx4f, dstf)
    su_a = s0ext[cin0 * npad:(cin0 + 1) * npad, 0:1]
    su_b = s0ext[(cin0 + 1) * npad:, 0:1]
    return _final_call(s2.reshape(ch, npad, F), dis_col, su_a, su_b,
                       W2, b2.reshape(1, -1), W3, b3.reshape(1, -1), npad, n)
